# baseline (speedup 1.0000x reference)
import functools

import jax
import jax.numpy as jnp
from jax import lax
from jax.experimental import pallas as pl
from jax.experimental.pallas import tpu as pltpu
from jax.experimental.pallas import tpu_sc as plsc

T = 2048
D = 1024
E_TR = 8
E_PAD = 16
CAP = 512
DFF = 4096
NC = 2
NS = 16
FFB = 1024
NF = DFF // FFB
NEG = -1e30


def _route_body(h_ref, wr_ref, w_ref):
    h = h_ref[...]
    wr = wr_ref[...]
    logits = lax.dot_general(wr, h, (((1,), (1,)), ((), ())),
                             preferred_element_type=jnp.float32)
    row = lax.broadcasted_iota(jnp.int32, (E_PAD, T), 0)
    logits = jnp.where(row < 9, logits, NEG)
    m1 = jnp.max(logits, axis=0, keepdims=True)
    is1 = logits == m1
    cnt1 = jnp.sum(is1.astype(jnp.int32), axis=0, keepdims=True)
    m2 = jnp.max(jnp.where(is1, NEG, logits), axis=0, keepdims=True)
    thr = jnp.where(cnt1 >= 2, m1, m2)
    top2 = logits >= thr
    p = jnp.exp(logits - m1)
    probs = p / jnp.sum(p, axis=0, keepdims=True)
    cand = top2 & (row < E_TR)
    key = jnp.where(cand, lax.bitcast_convert_type(probs, jnp.int32), -1)

    def bs(i, lohi):
        lo, hi = lohi
        mid = lo + (hi - lo) // 2
        cnt = jnp.sum((key > mid).astype(jnp.int32), axis=1, keepdims=True)
        gt = cnt > CAP
        return jnp.where(gt, mid + 1, lo), jnp.where(gt, hi, mid)

    lo0 = jnp.full((E_PAD, 1), -1, jnp.int32)
    hi0 = jnp.full((E_PAD, 1), 2**30, jnp.int32)
    _, tau = lax.fori_loop(0, 31, bs, (lo0, hi0))
    kept_strict = key > tau
    spots = CAP - jnp.sum(kept_strict.astype(jnp.int32), axis=1, keepdims=True)
    ties = cand & (key == tau)
    tok = lax.broadcasted_iota(jnp.int32, (E_PAD, T), 1)

    def bs2(i, lohi):
        lo, hi = lohi
        mid = (lo + hi + 1) // 2
        cnt = jnp.sum((ties & (tok < mid)).astype(jnp.int32), axis=1,
                      keepdims=True)
        ok = cnt <= spots
        return jnp.where(ok, mid, lo), jnp.where(ok, hi, mid - 1)

    sigma, _ = lax.fori_loop(0, 12, bs2, (jnp.zeros((E_PAD, 1), jnp.int32),
                                          jnp.full((E_PAD, 1), T, jnp.int32)))
    kept = kept_strict | (ties & (tok < sigma))
    w = jnp.where(kept, probs, 0.0)
    w_ref[...] = w[:E_TR]


def _route(h, wr_pad):
    return pl.pallas_call(
        _route_body,
        out_shape=jax.ShapeDtypeStruct((E_TR, T), jnp.float32),
    )(h, wr_pad)


def _compact_body(w_hbm, idx_hbm, gate_hbm, v_hbm, w_v, idx_v, gate_v, v_v):
    e = lax.axis_index("s") * NC + lax.axis_index("c")

    @pl.when(e < E_TR)
    def _():
        pltpu.sync_copy(w_hbm.at[e], w_v)
        zi = jnp.full((16,), T, jnp.int32)
        zf = jnp.zeros((16,), jnp.float32)
        for j in range(CAP // 16):
            idx_v[pl.ds(j * 16, 16)] = zi
            gate_v[pl.ds(j * 16, 16)] = zf

        def body(i, running):
            wv = w_v[pl.ds(i * 16, 16)]
            m = wv > 0.0
            mi = jnp.where(m, 1, 0).astype(jnp.int32)
            c = jnp.cumsum(mi)
            slot = running + c - 1
            slot = jnp.where(m, slot, 0)
            v_v[pl.ds(i * 16, 16)] = jnp.where(m, e * CAP + slot, E_TR * CAP)
            toks = lax.broadcasted_iota(jnp.int32, (16,), 0) + i * 16
            plsc.store_scatter(idx_v, [slot], toks, mask=m)
            plsc.store_scatter(gate_v, [slot], wv, mask=m)
            return running + jnp.sum(mi)

        lax.fori_loop(0, T // 16, body, jnp.int32(0))
        pltpu.sync_copy(idx_v, idx_hbm.at[e])
        pltpu.sync_copy(gate_v, gate_hbm.at[e])
        pltpu.sync_copy(v_v, v_hbm.at[e])


def _compact(w):
    return pl.kernel(
        _compact_body,
        out_type=[jax.ShapeDtypeStruct((E_TR, CAP), jnp.int32),
                  jax.ShapeDtypeStruct((E_TR, CAP), jnp.float32),
                  jax.ShapeDtypeStruct((E_TR, T), jnp.int32)],
        mesh=plsc.VectorSubcoreMesh(core_axis_name="c", subcore_axis_name="s",
                                    num_cores=NC, num_subcores=NS),
        scratch_types=[pltpu.VMEM((T,), jnp.float32),
                       pltpu.VMEM((CAP,), jnp.int32),
                       pltpu.VMEM((CAP,), jnp.float32),
                       pltpu.VMEM((T,), jnp.int32)],
        compiler_params=pltpu.CompilerParams(needs_layout_passes=False),
    )(w)


_GCH = 32
_GNC = 4


def _gather_body(h_hbm, idx2_hbm, xin_hbm, idx_v, r0, r1, r2,
                 sg0, sg1, sg2, sw0, sw1, sw2):
    wid = lax.axis_index("s") * NC + lax.axis_index("c")
    per_w = _GCH * _GNC
    pltpu.sync_copy(idx2_hbm.at[pl.ds(wid * _GNC, _GNC)], idx_v)
    for j in range(per_w // 16):
        v = idx_v[j // (_GCH // 16), pl.ds((j % (_GCH // 16)) * 16, 16)]
        idx_v[j // (_GCH // 16), pl.ds((j % (_GCH // 16)) * 16, 16)] = (
            jnp.minimum(v, T - 1))
    bufs = (r0, r1, r2)
    gsems = (sg0, sg1, sg2)
    wsems = (sw0, sw1, sw2)
    gds = [None] * _GNC
    wds = [None] * _GNC
    for ch in range(3):
        gds[ch] = pltpu.async_copy(h_hbm.at[idx_v.at[ch]], bufs[ch],
                                   gsems[ch])
    for ch in range(_GNC):
        p = ch % 3
        gds[ch].wait()
        b = wid * per_w + ch * _GCH
        wds[ch] = pltpu.async_copy(bufs[p], xin_hbm.at[pl.ds(b, _GCH)],
                                   wsems[p])
        if ch + 3 < _GNC:
            wds[ch].wait()
            gds[ch + 3] = pltpu.async_copy(h_hbm.at[idx_v.at[ch + 3]],
                                           bufs[p], gsems[p])
    for ch in range(max(0, _GNC - 3), _GNC):
        wds[ch].wait()


def _gather(h, idx2):
    return pl.kernel(
        _gather_body,
        out_type=jax.ShapeDtypeStruct((E_TR * CAP, D), jnp.float32),
        mesh=plsc.VectorSubcoreMesh(core_axis_name="c", subcore_axis_name="s",
                                    num_cores=NC, num_subcores=NS),
        scratch_types=[pltpu.VMEM((_GNC, _GCH), jnp.int32),
                       pltpu.VMEM((_GCH, D), jnp.float32),
                       pltpu.VMEM((_GCH, D), jnp.float32),
                       pltpu.VMEM((_GCH, D), jnp.float32),
                       pltpu.SemaphoreType.DMA, pltpu.SemaphoreType.DMA,
                       pltpu.SemaphoreType.DMA, pltpu.SemaphoreType.DMA,
                       pltpu.SemaphoreType.DMA, pltpu.SemaphoreType.DMA],
        compiler_params=pltpu.CompilerParams(needs_layout_passes=False),
    )(h, idx2)


def _mlp_body(xin_ref, w1_ref, w2_ref, gate_ref, out_ref, acc_ref):
    f = pl.program_id(1)
    x = xin_ref[0].astype(jnp.bfloat16)
    w1 = w1_ref[0].astype(jnp.bfloat16)
    hmid = lax.dot_general(x, w1, (((1,), (0,)), ((), ())),
                           preferred_element_type=jnp.float32)
    hmid = jax.nn.gelu(hmid)
    w2 = w2_ref[0].astype(jnp.bfloat16)
    part = lax.dot_general(hmid.astype(jnp.bfloat16), w2,
                           (((1,), (0,)), ((), ())),
                           preferred_element_type=jnp.float32)

    @pl.when(f == 0)
    def _():
        acc_ref[...] = part

    @pl.when(f != 0)
    def _():
        acc_ref[...] = acc_ref[...] + part

    @pl.when(f == NF - 1)
    def _():
        out_ref[0] = acc_ref[...] * gate_ref[0]


def _mlp(xin, w1, w2, gate):
    return pl.pallas_call(
        _mlp_body,
        grid=(E_TR, NF),
        in_specs=[
            pl.BlockSpec((1, CAP, D), lambda e, f: (e, 0, 0)),
            pl.BlockSpec((1, D, FFB), lambda e, f: (e, 0, f)),
            pl.BlockSpec((1, FFB, D), lambda e, f: (e, f, 0)),
            pl.BlockSpec((1, CAP, 1), lambda e, f: (e, 0, 0)),
        ],
        out_specs=pl.BlockSpec((1, CAP, D), lambda e, f: (e, 0, 0)),
        out_shape=jax.ShapeDtypeStruct((E_TR, CAP, D), jnp.float32),
        scratch_shapes=[pltpu.VMEM((CAP, D), jnp.float32)],
        compiler_params=pltpu.CompilerParams(
            dimension_semantics=("arbitrary", "arbitrary")),
    )(xin, w1, w2, gate)


_CCH = 16
_CNC = 4
_NROW = E_TR * CAP


def _combine_body(h_hbm, outs_hbm, vmap_hbm, y_hbm,
                  s01_v, v0_v, v1_v, va, ya, yb, ra, rb,
                  sv, sha, shb, sga, sgb, swa, swb):
    wid = lax.axis_index("s") * NC + lax.axis_index("c")
    per_w = _CCH * _CNC
    t0 = wid * per_w
    vds = []
    for e in range(E_TR):
        vds.append(pltpu.async_copy(vmap_hbm.at[e, pl.ds(t0, per_w)],
                                    va.at[e], sv))
    for d in vds:
        d.wait()
    for ch in range(_CNC):
        sl = pl.ds(ch * _CCH, 16)
        m1 = va[0, sl]
        for e in range(1, E_TR):
            m1 = jnp.minimum(m1, va[e, sl])
        m2 = jnp.full((16,), _NROW, jnp.int32)
        for e in range(E_TR):
            a = va[e, sl]
            m2 = jnp.minimum(m2, jnp.where(a == m1, _NROW, a))
        v0_v[ch, pl.ds(0, 16)] = jnp.where(m1 < _NROW, 1.0, 0.0)
        v1_v[ch, pl.ds(0, 16)] = jnp.where(m2 < _NROW, 1.0, 0.0)
        s01_v[ch, pl.ds(0, 16)] = jnp.minimum(m1, _NROW - 1)
        s01_v[ch, pl.ds(16, 16)] = jnp.minimum(m2, _NROW - 1)
    ybufs = (ya, yb)
    rbufs = (ra, rb)
    hsems = (sha, shb)
    gsems = (sga, sgb)
    wsems = (swa, swb)
    hds, gds, wds = [None] * 4, [None] * 4, [None] * 4
    for ch in range(2):
        hds[ch] = pltpu.async_copy(h_hbm.at[pl.ds(t0 + ch * _CCH, _CCH)],
                                   ybufs[ch], hsems[ch])
        gds[ch] = pltpu.async_copy(outs_hbm.at[s01_v.at[ch]],
                                   rbufs[ch], gsems[ch])
    for ch in range(_CNC):
        p = ch % 2
        hds[ch].wait()
        gds[ch].wait()
        yv, rv = ybufs[p], rbufs[p]
        val0 = v0_v[ch, pl.ds(0, 16)]
        val1 = v1_v[ch, pl.ds(0, 16)]
        for k in range(16):
            g0 = val0[k]
            g1 = val1[k]

            def dbody(i, _):
                for u in range(8):
                    sl2 = pl.ds(i * 128 + u * 16, 16)
                    yv[k, sl2] = (yv[k, sl2] + rv[k, sl2] * g0
                                  + rv[16 + k, sl2] * g1)
                return 0

            lax.fori_loop(0, D // 128, dbody, 0)
        wds[ch] = pltpu.async_copy(yv, y_hbm.at[pl.ds(t0 + ch * _CCH, _CCH)],
                                   wsems[p])
        if ch + 2 < _CNC:
            gds[ch + 2] = pltpu.async_copy(outs_hbm.at[s01_v.at[ch + 2]],
                                           rbufs[p], gsems[p])
            wds[ch].wait()
            hds[ch + 2] = pltpu.async_copy(
                h_hbm.at[pl.ds(t0 + (ch + 2) * _CCH, _CCH)], ybufs[p],
                hsems[p])
    wds[_CNC - 2].wait()
    wds[_CNC - 1].wait()


def _combine(h, outs_flat, vmap):
    return pl.kernel(
        _combine_body,
        out_type=jax.ShapeDtypeStruct((T, D), jnp.float32),
        mesh=plsc.VectorSubcoreMesh(core_axis_name="c", subcore_axis_name="s",
                                    num_cores=NC, num_subcores=NS),
        scratch_types=[pltpu.VMEM((_CNC, 2 * _CCH), jnp.int32),
                       pltpu.VMEM((_CNC, _CCH), jnp.float32),
                       pltpu.VMEM((_CNC, _CCH), jnp.float32),
                       pltpu.VMEM((E_TR, _CCH * _CNC), jnp.int32),
                       pltpu.VMEM((_CCH, D), jnp.float32),
                       pltpu.VMEM((_CCH, D), jnp.float32),
                       pltpu.VMEM((2 * _CCH, D), jnp.float32),
                       pltpu.VMEM((2 * _CCH, D), jnp.float32),
                       pltpu.SemaphoreType.DMA, pltpu.SemaphoreType.DMA,
                       pltpu.SemaphoreType.DMA, pltpu.SemaphoreType.DMA,
                       pltpu.SemaphoreType.DMA, pltpu.SemaphoreType.DMA,
                       pltpu.SemaphoreType.DMA],
        compiler_params=pltpu.CompilerParams(needs_layout_passes=False),
    )(h, outs_flat, vmap)


def kernel(h, Wr, W1, W2):
    wr_pad = jnp.zeros((E_PAD, D), jnp.float32).at[:Wr.shape[0]].set(Wr)
    w = _route(h, wr_pad)
    idx, gate, vmap = _compact(w)
    idx2 = idx.reshape(E_TR * CAP // _GCH, _GCH)
    xin = _gather(h, idx2)
    outs = _mlp(xin.reshape(E_TR, CAP, D), W1, W2,
                gate.reshape(E_TR, CAP, 1))
    return _combine(h, outs.reshape(E_TR * CAP, D), vmap)


# speedup vs baseline: 1.0203x; 1.0203x over previous
import functools

import jax
import jax.numpy as jnp
from jax import lax
from jax.experimental import pallas as pl
from jax.experimental.pallas import tpu as pltpu
from jax.experimental.pallas import tpu_sc as plsc

T = 2048
D = 1024
E_TR = 8
E_PAD = 16
CAP = 512
DFF = 4096
NC = 2
NS = 16
FFB = 1024
NF = DFF // FFB
NEG = -1e30


def _route_body(h_ref, wr_ref, w_ref):
    h = h_ref[...]
    wr = wr_ref[...]
    logits = lax.dot_general(wr, h, (((1,), (1,)), ((), ())),
                             preferred_element_type=jnp.float32)
    row = lax.broadcasted_iota(jnp.int32, (E_PAD, T), 0)
    logits = jnp.where(row < 9, logits, NEG)
    m1 = jnp.max(logits, axis=0, keepdims=True)
    is1 = logits == m1
    cnt1 = jnp.sum(is1.astype(jnp.int32), axis=0, keepdims=True)
    m2 = jnp.max(jnp.where(is1, NEG, logits), axis=0, keepdims=True)
    thr = jnp.where(cnt1 >= 2, m1, m2)
    top2 = logits >= thr
    p = jnp.exp(logits - m1)
    probs = p / jnp.sum(p, axis=0, keepdims=True)
    cand = top2 & (row < E_TR)
    key = jnp.where(cand, lax.bitcast_convert_type(probs, jnp.int32), -1)

    def bs(i, lohi):
        lo, hi = lohi
        mid = lo + (hi - lo) // 2
        cnt = jnp.sum((key > mid).astype(jnp.int32), axis=1, keepdims=True)
        gt = cnt > CAP
        return jnp.where(gt, mid + 1, lo), jnp.where(gt, hi, mid)

    lo0 = jnp.full((E_PAD, 1), -1, jnp.int32)
    hi0 = jnp.full((E_PAD, 1), 2**30, jnp.int32)
    _, tau = lax.fori_loop(0, 31, bs, (lo0, hi0))
    kept_strict = key > tau
    spots = CAP - jnp.sum(kept_strict.astype(jnp.int32), axis=1, keepdims=True)
    ties = cand & (key == tau)
    tok = lax.broadcasted_iota(jnp.int32, (E_PAD, T), 1)

    def bs2(i, lohi):
        lo, hi = lohi
        mid = (lo + hi + 1) // 2
        cnt = jnp.sum((ties & (tok < mid)).astype(jnp.int32), axis=1,
                      keepdims=True)
        ok = cnt <= spots
        return jnp.where(ok, mid, lo), jnp.where(ok, hi, mid - 1)

    sigma, _ = lax.fori_loop(0, 12, bs2, (jnp.zeros((E_PAD, 1), jnp.int32),
                                          jnp.full((E_PAD, 1), T, jnp.int32)))
    kept = kept_strict | (ties & (tok < sigma))
    w = jnp.where(kept, probs, 0.0)
    w_ref[...] = w[:E_TR]


def _route(h, wr_pad):
    return pl.pallas_call(
        _route_body,
        out_shape=jax.ShapeDtypeStruct((E_TR, T), jnp.float32),
    )(h, wr_pad)


def _compact_body(w_hbm, idx_hbm, gate_hbm, v_hbm, w_v, idx_v, gate_v, v_v):
    e = lax.axis_index("s") * NC + lax.axis_index("c")

    @pl.when(e < E_TR)
    def _():
        pltpu.sync_copy(w_hbm.at[e], w_v)
        zi = jnp.full((16,), T, jnp.int32)
        zf = jnp.zeros((16,), jnp.float32)
        for j in range(CAP // 16):
            idx_v[pl.ds(j * 16, 16)] = zi
            gate_v[pl.ds(j * 16, 16)] = zf

        def body(i, running):
            wv = w_v[pl.ds(i * 16, 16)]
            m = wv > 0.0
            mi = jnp.where(m, 1, 0).astype(jnp.int32)
            c = jnp.cumsum(mi)
            slot = running + c - 1
            slot = jnp.where(m, slot, 0)
            v_v[pl.ds(i * 16, 16)] = jnp.where(m, e * CAP + slot, E_TR * CAP)
            toks = lax.broadcasted_iota(jnp.int32, (16,), 0) + i * 16
            plsc.store_scatter(idx_v, [slot], toks, mask=m)
            plsc.store_scatter(gate_v, [slot], wv, mask=m)
            return running + jnp.sum(mi)

        lax.fori_loop(0, T // 16, body, jnp.int32(0))
        pltpu.sync_copy(idx_v, idx_hbm.at[e])
        pltpu.sync_copy(gate_v, gate_hbm.at[e])
        pltpu.sync_copy(v_v, v_hbm.at[e])


def _compact(w):
    return pl.kernel(
        _compact_body,
        out_type=[jax.ShapeDtypeStruct((E_TR, CAP), jnp.int32),
                  jax.ShapeDtypeStruct((E_TR, CAP), jnp.float32),
                  jax.ShapeDtypeStruct((E_TR, T), jnp.int32)],
        mesh=plsc.VectorSubcoreMesh(core_axis_name="c", subcore_axis_name="s",
                                    num_cores=NC, num_subcores=NS),
        scratch_types=[pltpu.VMEM((T,), jnp.float32),
                       pltpu.VMEM((CAP,), jnp.int32),
                       pltpu.VMEM((CAP,), jnp.float32),
                       pltpu.VMEM((T,), jnp.int32)],
        compiler_params=pltpu.CompilerParams(needs_layout_passes=False),
    )(w)


_GCH = 32
_GNC = 4


def _gather_body(h_hbm, idx2_hbm, xin_hbm, idx_v, r0, r1, sg0, sg1, sw0, sw1):
    wid = lax.axis_index("s") * NC + lax.axis_index("c")
    per_w = _GCH * _GNC
    pltpu.sync_copy(idx2_hbm.at[pl.ds(wid * _GNC, _GNC)], idx_v)
    for j in range(per_w // 16):
        v = idx_v[j // (_GCH // 16), pl.ds((j % (_GCH // 16)) * 16, 16)]
        idx_v[j // (_GCH // 16), pl.ds((j % (_GCH // 16)) * 16, 16)] = (
            jnp.minimum(v, T - 1))
    bufs = (r0, r1)
    gsems = (sg0, sg1)
    wsems = (sw0, sw1)
    g0 = pltpu.async_copy(h_hbm.at[idx_v.at[0]], r0, sg0)
    g1 = pltpu.async_copy(h_hbm.at[idx_v.at[1]], r1, sg1)
    gds = [g0, g1, None, None]
    wds = [None, None, None, None]
    for ch in range(_GNC):
        gds[ch].wait()
        b = wid * per_w + ch * _GCH
        wds[ch] = pltpu.async_copy(bufs[ch % 2], xin_hbm.at[pl.ds(b, _GCH)],
                                   wsems[ch % 2])
        if ch + 2 < _GNC:
            wds[ch].wait()
            gds[ch + 2] = pltpu.async_copy(h_hbm.at[idx_v.at[ch + 2]],
                                           bufs[ch % 2], gsems[ch % 2])
    wds[_GNC - 2].wait()
    wds[_GNC - 1].wait()


def _gather(h, idx2):
    return pl.kernel(
        _gather_body,
        out_type=jax.ShapeDtypeStruct((E_TR * CAP, D), jnp.float32),
        mesh=plsc.VectorSubcoreMesh(core_axis_name="c", subcore_axis_name="s",
                                    num_cores=NC, num_subcores=NS),
        scratch_types=[pltpu.VMEM((_GNC, _GCH), jnp.int32),
                       pltpu.VMEM((_GCH, D), jnp.float32),
                       pltpu.VMEM((_GCH, D), jnp.float32),
                       pltpu.SemaphoreType.DMA, pltpu.SemaphoreType.DMA,
                       pltpu.SemaphoreType.DMA, pltpu.SemaphoreType.DMA],
        compiler_params=pltpu.CompilerParams(needs_layout_passes=False),
    )(h, idx2)


def _mlp_body(xin_ref, w1_ref, w2_ref, gate_ref, out_ref, acc_ref):
    f = pl.program_id(1)
    x = xin_ref[0].astype(jnp.bfloat16)
    w1 = w1_ref[0].astype(jnp.bfloat16)
    hmid = lax.dot_general(x, w1, (((1,), (0,)), ((), ())),
                           preferred_element_type=jnp.float32)
    hmid = jax.nn.gelu(hmid)
    w2 = w2_ref[0].astype(jnp.bfloat16)
    part = lax.dot_general(hmid.astype(jnp.bfloat16), w2,
                           (((1,), (0,)), ((), ())),
                           preferred_element_type=jnp.float32)

    @pl.when(f == 0)
    def _():
        acc_ref[...] = part

    @pl.when(f != 0)
    def _():
        acc_ref[...] = acc_ref[...] + part

    @pl.when(f == NF - 1)
    def _():
        out_ref[0] = acc_ref[...] * gate_ref[0]


def _mlp(xin, w1, w2, gate):
    return pl.pallas_call(
        _mlp_body,
        grid=(E_TR, NF),
        in_specs=[
            pl.BlockSpec((1, CAP, D), lambda e, f: (e, 0, 0)),
            pl.BlockSpec((1, D, FFB), lambda e, f: (e, 0, f)),
            pl.BlockSpec((1, FFB, D), lambda e, f: (e, f, 0)),
            pl.BlockSpec((1, CAP, 1), lambda e, f: (e, 0, 0)),
        ],
        out_specs=pl.BlockSpec((1, CAP, D), lambda e, f: (e, 0, 0)),
        out_shape=jax.ShapeDtypeStruct((E_TR, CAP, D), jnp.float32),
        scratch_shapes=[pltpu.VMEM((CAP, D), jnp.float32)],
        compiler_params=pltpu.CompilerParams(
            dimension_semantics=("arbitrary", "arbitrary")),
    )(xin, w1, w2, gate)


_CCH = 16
_CNC = 4
_NROW = E_TR * CAP


def _combine_body(h_hbm, outs_hbm, vmap_hbm, y_hbm,
                  s01_v, v0_v, v1_v, va, ya, yb, ra, rb,
                  sv, sha, shb, sga, sgb, swa, swb):
    wid = lax.axis_index("s") * NC + lax.axis_index("c")
    per_w = _CCH * _CNC
    t0 = wid * per_w
    vds = []
    for e in range(E_TR):
        vds.append(pltpu.async_copy(vmap_hbm.at[e, pl.ds(t0, per_w)],
                                    va.at[e], sv))
    for d in vds:
        d.wait()
    for ch in range(_CNC):
        sl = pl.ds(ch * _CCH, 16)
        m1 = va[0, sl]
        for e in range(1, E_TR):
            m1 = jnp.minimum(m1, va[e, sl])
        m2 = jnp.full((16,), _NROW, jnp.int32)
        for e in range(E_TR):
            a = va[e, sl]
            m2 = jnp.minimum(m2, jnp.where(a == m1, _NROW, a))
        v0_v[ch, pl.ds(0, 16)] = jnp.where(m1 < _NROW, 1.0, 0.0)
        v1_v[ch, pl.ds(0, 16)] = jnp.where(m2 < _NROW, 1.0, 0.0)
        s01_v[ch, pl.ds(0, 16)] = jnp.minimum(m1, _NROW - 1)
        s01_v[ch, pl.ds(16, 16)] = jnp.minimum(m2, _NROW - 1)
    ybufs = (ya, yb)
    rbufs = (ra, rb)
    hsems = (sha, shb)
    gsems = (sga, sgb)
    wsems = (swa, swb)
    hds, gds, wds = [None] * 4, [None] * 4, [None] * 4
    for ch in range(2):
        hds[ch] = pltpu.async_copy(h_hbm.at[pl.ds(t0 + ch * _CCH, _CCH)],
                                   ybufs[ch], hsems[ch])
        gds[ch] = pltpu.async_copy(outs_hbm.at[s01_v.at[ch]],
                                   rbufs[ch], gsems[ch])
    for ch in range(_CNC):
        p = ch % 2
        hds[ch].wait()
        gds[ch].wait()
        yv, rv = ybufs[p], rbufs[p]
        val0 = v0_v[ch, pl.ds(0, 16)]
        val1 = v1_v[ch, pl.ds(0, 16)]
        for k in range(16):
            g0 = val0[k]
            g1 = val1[k]

            def dbody(i, _):
                for u in range(8):
                    sl2 = pl.ds(i * 128 + u * 16, 16)
                    yv[k, sl2] = (yv[k, sl2] + rv[k, sl2] * g0
                                  + rv[16 + k, sl2] * g1)
                return 0

            lax.fori_loop(0, D // 128, dbody, 0)
        wds[ch] = pltpu.async_copy(yv, y_hbm.at[pl.ds(t0 + ch * _CCH, _CCH)],
                                   wsems[p])
        if ch + 2 < _CNC:
            gds[ch + 2] = pltpu.async_copy(outs_hbm.at[s01_v.at[ch + 2]],
                                           rbufs[p], gsems[p])
            wds[ch].wait()
            hds[ch + 2] = pltpu.async_copy(
                h_hbm.at[pl.ds(t0 + (ch + 2) * _CCH, _CCH)], ybufs[p],
                hsems[p])
    wds[_CNC - 2].wait()
    wds[_CNC - 1].wait()


def _combine(h, outs_flat, vmap):
    return pl.kernel(
        _combine_body,
        out_type=jax.ShapeDtypeStruct((T, D), jnp.float32),
        mesh=plsc.VectorSubcoreMesh(core_axis_name="c", subcore_axis_name="s",
                                    num_cores=NC, num_subcores=NS),
        scratch_types=[pltpu.VMEM((_CNC, 2 * _CCH), jnp.int32),
                       pltpu.VMEM((_CNC, _CCH), jnp.float32),
                       pltpu.VMEM((_CNC, _CCH), jnp.float32),
                       pltpu.VMEM((E_TR, _CCH * _CNC), jnp.int32),
                       pltpu.VMEM((_CCH, D), jnp.float32),
                       pltpu.VMEM((_CCH, D), jnp.float32),
                       pltpu.VMEM((2 * _CCH, D), jnp.float32),
                       pltpu.VMEM((2 * _CCH, D), jnp.float32),
                       pltpu.SemaphoreType.DMA, pltpu.SemaphoreType.DMA,
                       pltpu.SemaphoreType.DMA, pltpu.SemaphoreType.DMA,
                       pltpu.SemaphoreType.DMA, pltpu.SemaphoreType.DMA,
                       pltpu.SemaphoreType.DMA],
        compiler_params=pltpu.CompilerParams(needs_layout_passes=False),
    )(h, outs_flat, vmap)


def kernel(h, Wr, W1, W2):
    wr_pad = jnp.zeros((E_PAD, D), jnp.float32).at[:Wr.shape[0]].set(Wr)
    w = _route(h, wr_pad)
    idx, gate, vmap = _compact(w)
    idx2 = idx.reshape(E_TR * CAP // _GCH, _GCH)
    xin = _gather(h, idx2)
    outs = _mlp(xin.reshape(E_TR, CAP, D), W1, W2,
                gate.reshape(E_TR, CAP, 1))
    return _combine(h, outs.reshape(E_TR * CAP, D), vmap)


# speedup vs baseline: 1.0205x; 1.0002x over previous
import jax
import jax.numpy as jnp
from jax import lax
from jax.experimental import pallas as pl
from jax.experimental.pallas import tpu as pltpu
from jax.experimental.pallas import tpu_sc as plsc

T = 2048
D = 1024
E_TR = 8
E_PAD = 16
CAP = 512
DFF = 4096
NC = 2
NS = 16
FFB = 1024
NF = DFF // FFB
NEG = -1e30


def _route_body(h_ref, wr_ref, w_ref):
    h = h_ref[...]
    wr = wr_ref[...]
    logits = lax.dot_general(wr, h, (((1,), (1,)), ((), ())),
                             preferred_element_type=jnp.float32)
    row = lax.broadcasted_iota(jnp.int32, (E_PAD, T), 0)
    logits = jnp.where(row < 9, logits, NEG)
    m1 = jnp.max(logits, axis=0, keepdims=True)
    is1 = logits == m1
    cnt1 = jnp.sum(is1.astype(jnp.int32), axis=0, keepdims=True)
    m2 = jnp.max(jnp.where(is1, NEG, logits), axis=0, keepdims=True)
    thr = jnp.where(cnt1 >= 2, m1, m2)
    top2 = logits >= thr
    p = jnp.exp(logits - m1)
    probs = p / jnp.sum(p, axis=0, keepdims=True)
    cand = top2 & (row < E_TR)
    key = jnp.where(cand, lax.bitcast_convert_type(probs, jnp.int32), -1)

    def bs(i, lohi):
        lo, hi = lohi
        mid = lo + (hi - lo) // 2
        cnt = jnp.sum((key > mid).astype(jnp.int32), axis=1, keepdims=True)
        gt = cnt > CAP
        return jnp.where(gt, mid + 1, lo), jnp.where(gt, hi, mid)

    lo0 = jnp.full((E_PAD, 1), -1, jnp.int32)
    hi0 = jnp.full((E_PAD, 1), 2**30, jnp.int32)
    _, tau = lax.fori_loop(0, 31, bs, (lo0, hi0))
    kept_strict = key > tau
    spots = CAP - jnp.sum(kept_strict.astype(jnp.int32), axis=1, keepdims=True)
    ties = cand & (key == tau)
    tok = lax.broadcasted_iota(jnp.int32, (E_PAD, T), 1)

    def bs2(i, lohi):
        lo, hi = lohi
        mid = (lo + hi + 1) // 2
        cnt = jnp.sum((ties & (tok < mid)).astype(jnp.int32), axis=1,
                      keepdims=True)
        ok = cnt <= spots
        return jnp.where(ok, mid, lo), jnp.where(ok, hi, mid - 1)

    sigma, _ = lax.fori_loop(0, 12, bs2, (jnp.zeros((E_PAD, 1), jnp.int32),
                                          jnp.full((E_PAD, 1), T, jnp.int32)))
    kept = kept_strict | (ties & (tok < sigma))
    w = jnp.where(kept, probs, 0.0)
    w_ref[...] = w[:E_TR]


def _route(h, wr_pad):
    return pl.pallas_call(
        _route_body,
        out_shape=jax.ShapeDtypeStruct((E_TR, T), jnp.float32),
    )(h, wr_pad)


def _compact_body(w_hbm, idx_hbm, gate_hbm, v_hbm, w_v, idx_v, gate_v, v_v):
    e = lax.axis_index("s") * NC + lax.axis_index("c")

    @pl.when(e < E_TR)
    def _():
        pltpu.sync_copy(w_hbm.at[e], w_v)
        zi = jnp.full((16,), T, jnp.int32)
        zf = jnp.zeros((16,), jnp.float32)
        for j in range(CAP // 16):
            idx_v[pl.ds(j * 16, 16)] = zi
            gate_v[pl.ds(j * 16, 16)] = zf

        def body(i, running):
            wv = w_v[pl.ds(i * 16, 16)]
            m = wv > 0.0
            mi = jnp.where(m, 1, 0).astype(jnp.int32)
            c = jnp.cumsum(mi)
            slot = running + c - 1
            slot = jnp.where(m, slot, 0)
            v_v[pl.ds(i * 16, 16)] = jnp.where(m, e * CAP + slot, E_TR * CAP)
            toks = lax.broadcasted_iota(jnp.int32, (16,), 0) + i * 16
            plsc.store_scatter(idx_v, [slot], toks, mask=m)
            plsc.store_scatter(gate_v, [slot], wv, mask=m)
            return running + jnp.sum(mi)

        lax.fori_loop(0, T // 16, body, jnp.int32(0))
        pltpu.sync_copy(idx_v, idx_hbm.at[e])
        pltpu.sync_copy(gate_v, gate_hbm.at[e])
        pltpu.sync_copy(v_v, v_hbm.at[e])


def _compact(w):
    return pl.kernel(
        _compact_body,
        out_type=[jax.ShapeDtypeStruct((E_TR, CAP), jnp.int32),
                  jax.ShapeDtypeStruct((E_TR, CAP), jnp.float32),
                  jax.ShapeDtypeStruct((E_TR, T), jnp.int32)],
        mesh=plsc.VectorSubcoreMesh(core_axis_name="c", subcore_axis_name="s",
                                    num_cores=NC, num_subcores=NS),
        scratch_types=[pltpu.VMEM((T,), jnp.float32),
                       pltpu.VMEM((CAP,), jnp.int32),
                       pltpu.VMEM((CAP,), jnp.float32),
                       pltpu.VMEM((T,), jnp.int32)],
        compiler_params=pltpu.CompilerParams(needs_layout_passes=False),
    )(w)


_GCH = 32
_GNC = 4


def _gather_body(h_hbm, idx2_hbm, xin_hbm, idx_v, r0, r1, sg0, sg1, sw0, sw1):
    wid = lax.axis_index("s") * NC + lax.axis_index("c")
    per_w = _GCH * _GNC
    pltpu.sync_copy(idx2_hbm.at[pl.ds(wid * _GNC, _GNC)], idx_v)
    for j in range(per_w // 16):
        v = idx_v[j // (_GCH // 16), pl.ds((j % (_GCH // 16)) * 16, 16)]
        idx_v[j // (_GCH // 16), pl.ds((j % (_GCH // 16)) * 16, 16)] = (
            jnp.minimum(v, T - 1))
    bufs = (r0, r1)
    gsems = (sg0, sg1)
    wsems = (sw0, sw1)
    g0 = pltpu.async_copy(h_hbm.at[idx_v.at[0]], r0, sg0)
    g1 = pltpu.async_copy(h_hbm.at[idx_v.at[1]], r1, sg1)
    gds = [g0, g1, None, None]
    wds = [None, None, None, None]
    for ch in range(_GNC):
        gds[ch].wait()
        b = wid * per_w + ch * _GCH
        wds[ch] = pltpu.async_copy(bufs[ch % 2], xin_hbm.at[pl.ds(b, _GCH)],
                                   wsems[ch % 2])
        if ch + 2 < _GNC:
            wds[ch].wait()
            gds[ch + 2] = pltpu.async_copy(h_hbm.at[idx_v.at[ch + 2]],
                                           bufs[ch % 2], gsems[ch % 2])
    wds[_GNC - 2].wait()
    wds[_GNC - 1].wait()


def _gather(h, idx2):
    return pl.kernel(
        _gather_body,
        out_type=jax.ShapeDtypeStruct((E_TR * CAP, D), jnp.float32),
        mesh=plsc.VectorSubcoreMesh(core_axis_name="c", subcore_axis_name="s",
                                    num_cores=NC, num_subcores=NS),
        scratch_types=[pltpu.VMEM((_GNC, _GCH), jnp.int32),
                       pltpu.VMEM((_GCH, D), jnp.float32),
                       pltpu.VMEM((_GCH, D), jnp.float32),
                       pltpu.SemaphoreType.DMA, pltpu.SemaphoreType.DMA,
                       pltpu.SemaphoreType.DMA, pltpu.SemaphoreType.DMA],
        compiler_params=pltpu.CompilerParams(needs_layout_passes=False),
    )(h, idx2)


def _mlp_body(xin_ref, w1_ref, w2_ref, gate_ref, out_ref, acc_ref):
    f = pl.program_id(1)
    x = xin_ref[0].astype(jnp.bfloat16)
    w1 = w1_ref[0].astype(jnp.bfloat16)
    hmid = lax.dot_general(x, w1, (((1,), (0,)), ((), ())),
                           preferred_element_type=jnp.float32)
    hmid = jax.nn.gelu(hmid)
    w2 = w2_ref[0].astype(jnp.bfloat16)
    part = lax.dot_general(hmid.astype(jnp.bfloat16), w2,
                           (((1,), (0,)), ((), ())),
                           preferred_element_type=jnp.float32)

    @pl.when(f == 0)
    def _():
        acc_ref[...] = part

    @pl.when(f != 0)
    def _():
        acc_ref[...] = acc_ref[...] + part

    @pl.when(f == NF - 1)
    def _():
        out_ref[0] = acc_ref[...] * gate_ref[0]


def _mlp(xin, w1, w2, gate):
    return pl.pallas_call(
        _mlp_body,
        grid=(E_TR, NF),
        in_specs=[
            pl.BlockSpec((1, CAP, D), lambda e, f: (e, 0, 0)),
            pl.BlockSpec((1, D, FFB), lambda e, f: (e, 0, f)),
            pl.BlockSpec((1, FFB, D), lambda e, f: (e, f, 0)),
            pl.BlockSpec((1, CAP, 1), lambda e, f: (e, 0, 0)),
        ],
        out_specs=pl.BlockSpec((1, CAP, D), lambda e, f: (e, 0, 0)),
        out_shape=jax.ShapeDtypeStruct((E_TR, CAP, D), jnp.float32),
        scratch_shapes=[pltpu.VMEM((CAP, D), jnp.float32)],
        compiler_params=pltpu.CompilerParams(
            dimension_semantics=("arbitrary", "arbitrary")),
    )(xin, w1, w2, gate)


_CCH = 16
_CNC = 4
_NROW = E_TR * CAP


def _combine_body(h_hbm, outs_hbm, vmap_hbm, y_hbm,
                  s01_v, v0_v, v1_v, va, ya, yb, ra, rb,
                  sv, sha, shb, sga, sgb, swa, swb):
    wid = lax.axis_index("s") * NC + lax.axis_index("c")
    per_w = _CCH * _CNC
    t0 = wid * per_w
    vds = []
    for e in range(E_TR):
        vds.append(pltpu.async_copy(vmap_hbm.at[e, pl.ds(t0, per_w)],
                                    va.at[e], sv))
    for d in vds:
        d.wait()
    for ch in range(_CNC):
        sl = pl.ds(ch * _CCH, 16)
        m1 = va[0, sl]
        for e in range(1, E_TR):
            m1 = jnp.minimum(m1, va[e, sl])
        m2 = jnp.full((16,), _NROW, jnp.int32)
        for e in range(E_TR):
            a = va[e, sl]
            m2 = jnp.minimum(m2, jnp.where(a == m1, _NROW, a))
        v0_v[ch, pl.ds(0, 16)] = jnp.where(m1 < _NROW, 1.0, 0.0)
        v1_v[ch, pl.ds(0, 16)] = jnp.where(m2 < _NROW, 1.0, 0.0)
        s01_v[ch, pl.ds(0, 16)] = jnp.minimum(m1, _NROW - 1)
        s01_v[ch, pl.ds(16, 16)] = jnp.minimum(m2, _NROW - 1)
    ybufs = (ya, yb)
    rbufs = (ra, rb)
    hsems = (sha, shb)
    gsems = (sga, sgb)
    wsems = (swa, swb)
    hds, gds, wds = [None] * 4, [None] * 4, [None] * 4
    for ch in range(2):
        hds[ch] = pltpu.async_copy(h_hbm.at[pl.ds(t0 + ch * _CCH, _CCH)],
                                   ybufs[ch], hsems[ch])
        gds[ch] = pltpu.async_copy(outs_hbm.at[s01_v.at[ch]],
                                   rbufs[ch], gsems[ch])
    for ch in range(_CNC):
        p = ch % 2
        hds[ch].wait()
        gds[ch].wait()
        yv, rv = ybufs[p], rbufs[p]
        val0 = v0_v[ch, pl.ds(0, 16)]
        val1 = v1_v[ch, pl.ds(0, 16)]
        for k in range(16):
            g0 = val0[k]
            g1 = val1[k]

            def dbody(i, _):
                for u in range(8):
                    sl2 = pl.ds(i * 128 + u * 16, 16)
                    yv[k, sl2] = (yv[k, sl2] + rv[k, sl2] * g0
                                  + rv[16 + k, sl2] * g1)
                return 0

            lax.fori_loop(0, D // 128, dbody, 0)
        wds[ch] = pltpu.async_copy(yv, y_hbm.at[pl.ds(t0 + ch * _CCH, _CCH)],
                                   wsems[p])
        if ch + 2 < _CNC:
            gds[ch + 2] = pltpu.async_copy(outs_hbm.at[s01_v.at[ch + 2]],
                                           rbufs[p], gsems[p])
            wds[ch].wait()
            hds[ch + 2] = pltpu.async_copy(
                h_hbm.at[pl.ds(t0 + (ch + 2) * _CCH, _CCH)], ybufs[p],
                hsems[p])
    wds[_CNC - 2].wait()
    wds[_CNC - 1].wait()


def _combine(h, outs_flat, vmap):
    return pl.kernel(
        _combine_body,
        out_type=jax.ShapeDtypeStruct((T, D), jnp.float32),
        mesh=plsc.VectorSubcoreMesh(core_axis_name="c", subcore_axis_name="s",
                                    num_cores=NC, num_subcores=NS),
        scratch_types=[pltpu.VMEM((_CNC, 2 * _CCH), jnp.int32),
                       pltpu.VMEM((_CNC, _CCH), jnp.float32),
                       pltpu.VMEM((_CNC, _CCH), jnp.float32),
                       pltpu.VMEM((E_TR, _CCH * _CNC), jnp.int32),
                       pltpu.VMEM((_CCH, D), jnp.float32),
                       pltpu.VMEM((_CCH, D), jnp.float32),
                       pltpu.VMEM((2 * _CCH, D), jnp.float32),
                       pltpu.VMEM((2 * _CCH, D), jnp.float32),
                       pltpu.SemaphoreType.DMA, pltpu.SemaphoreType.DMA,
                       pltpu.SemaphoreType.DMA, pltpu.SemaphoreType.DMA,
                       pltpu.SemaphoreType.DMA, pltpu.SemaphoreType.DMA,
                       pltpu.SemaphoreType.DMA],
        compiler_params=pltpu.CompilerParams(needs_layout_passes=False),
    )(h, outs_flat, vmap)


def kernel(h, Wr, W1, W2):
    wr_pad = jnp.zeros((E_PAD, D), jnp.float32).at[:Wr.shape[0]].set(Wr)
    w = _route(h, wr_pad)
    idx, gate, vmap = _compact(w)
    idx2 = idx.reshape(E_TR * CAP // _GCH, _GCH)
    xin = _gather(h, idx2)
    outs = _mlp(xin.reshape(E_TR, CAP, D), W1, W2,
                gate.reshape(E_TR, CAP, 1))
    return _combine(h, outs.reshape(E_TR * CAP, D), vmap)
